# initial kernel scaffold (unmeasured)
import jax
import jax.numpy as jnp
from jax import lax
from jax.experimental import pallas as pl
from jax.experimental.pallas import tpu as pltpu

N_DEV = 16
H_PER = 8
SQ = 256
SKV = 4096
DH = 128
D_MODEL = 1024
RPC = SQ // N_DEV
SCALE = 0.08838834764831843


def kernel(x, Wq, Wo, K_ext, V_ext):
    x2 = x.reshape(SQ, D_MODEL)

    def body(x_ref, wq_ref, wo_ref, k_hbm, v_hbm, out_ref,
             k_buf, v_buf, attn_ref, partial_ref, recv_ref,
             kv_sems, a_send, a_recv, c_send, c_recv):
        my = lax.axis_index("i")

        barrier = pltpu.get_barrier_semaphore()
        for d in range(1, N_DEV):
            peer = (my + d) % N_DEV
            pl.semaphore_signal(barrier, inc=1, device_id=(peer,),
                                device_id_type=pl.DeviceIdType.MESH)
        pl.semaphore_wait(barrier, N_DEV - 1)

        h0 = my * H_PER
        k_copy = pltpu.make_async_copy(
            k_hbm.at[0, :, pl.ds(h0, H_PER), :], k_buf, kv_sems.at[0])
        v_copy = pltpu.make_async_copy(
            v_hbm.at[0, :, pl.ds(h0, H_PER), :], v_buf, kv_sems.at[1])
        k_copy.start()
        v_copy.start()

        xb = x_ref[...].astype(jnp.bfloat16)
        wqb = wq_ref[...].astype(jnp.bfloat16)
        q = lax.dot_general(xb, wqb, (((1,), (0,)), ((), ())),
                            preferred_element_type=jnp.float32)
        qb = (q * SCALE).astype(jnp.bfloat16)

        k_copy.wait()
        v_copy.wait()

        for h in range(H_PER):
            qh = qb[:, h * DH:(h + 1) * DH]
            kh = k_buf[:, h, :].astype(jnp.bfloat16)
            s = lax.dot_general(qh, kh, (((1,), (1,)), ((), ())),
                                preferred_element_type=jnp.float32)
            m = jnp.max(s, axis=1, keepdims=True)
            p = jnp.exp(s - m)
            l = jnp.sum(p, axis=1, keepdims=True)
            vh = v_buf[:, h, :].astype(jnp.bfloat16)
            o = lax.dot_general(p.astype(jnp.bfloat16), vh,
                                (((1,), (0,)), ((), ())),
                                preferred_element_type=jnp.float32)
            attn_ref[:, h * DH:(h + 1) * DH] = (o / l).astype(jnp.bfloat16)

        wob = wo_ref[...].astype(jnp.bfloat16)
        partial_ref[...] = lax.dot_general(
            attn_ref[...], wob, (((1,), (0,)), ((), ())),
            preferred_element_type=jnp.float32)

        a_sends = []
        for d in range(1, N_DEV):
            peer = (my + d) % N_DEV
            rdma = pltpu.make_async_remote_copy(
                src_ref=partial_ref.at[pl.ds(peer * RPC, RPC), :],
                dst_ref=recv_ref.at[my],
                send_sem=a_send.at[peer],
                recv_sem=a_recv.at[my],
                device_id=(peer,),
                device_id_type=pl.DeviceIdType.MESH,
            )
            rdma.start()
            a_sends.append(rdma)
        for d in range(1, N_DEV):
            src = (my + d) % N_DEV
            pltpu.make_async_remote_copy(
                src_ref=partial_ref.at[pl.ds(0, RPC), :],
                dst_ref=recv_ref.at[src],
                send_sem=a_send.at[src],
                recv_sem=a_recv.at[src],
                device_id=(src,),
                device_id_type=pl.DeviceIdType.MESH,
            ).wait_recv()
        for r in a_sends:
            r.wait_send()

        acc = partial_ref[pl.ds(my * RPC, RPC), :]
        for d in range(1, N_DEV):
            src = (my + d) % N_DEV
            acc = acc + recv_ref[src]
        out_ref[pl.ds(my * RPC, RPC), :] = acc

        c_sends = []
        for d in range(1, N_DEV):
            peer = (my + d) % N_DEV
            rdma = pltpu.make_async_remote_copy(
                src_ref=out_ref.at[pl.ds(my * RPC, RPC), :],
                dst_ref=out_ref.at[pl.ds(my * RPC, RPC), :],
                send_sem=c_send.at[peer],
                recv_sem=c_recv.at[my],
                device_id=(peer,),
                device_id_type=pl.DeviceIdType.MESH,
            )
            rdma.start()
            c_sends.append(rdma)
        for d in range(1, N_DEV):
            src = (my + d) % N_DEV
            pltpu.make_async_remote_copy(
                src_ref=out_ref.at[pl.ds(0, RPC), :],
                dst_ref=out_ref.at[pl.ds(src * RPC, RPC), :],
                send_sem=c_send.at[src],
                recv_sem=c_recv.at[src],
                device_id=(src,),
                device_id_type=pl.DeviceIdType.MESH,
            ).wait_recv()
        for r in c_sends:
            r.wait_send()

    out = pl.pallas_call(
        body,
        out_shape=jax.ShapeDtypeStruct((SQ, D_MODEL), jnp.float32),
        in_specs=[
            pl.BlockSpec(memory_space=pltpu.VMEM),
            pl.BlockSpec(memory_space=pltpu.VMEM),
            pl.BlockSpec(memory_space=pltpu.VMEM),
            pl.BlockSpec(memory_space=pltpu.ANY),
            pl.BlockSpec(memory_space=pltpu.ANY),
        ],
        out_specs=pl.BlockSpec(memory_space=pltpu.VMEM),
        scratch_shapes=[
            pltpu.VMEM((SKV, H_PER, DH), jnp.float32),
            pltpu.VMEM((SKV, H_PER, DH), jnp.float32),
            pltpu.VMEM((SQ, D_MODEL), jnp.bfloat16),
            pltpu.VMEM((SQ, D_MODEL), jnp.float32),
            pltpu.VMEM((N_DEV, RPC, D_MODEL), jnp.float32),
            pltpu.SemaphoreType.DMA((2,)),
            pltpu.SemaphoreType.DMA((N_DEV,)),
            pltpu.SemaphoreType.DMA((N_DEV,)),
            pltpu.SemaphoreType.DMA((N_DEV,)),
            pltpu.SemaphoreType.DMA((N_DEV,)),
        ],
        compiler_params=pltpu.CompilerParams(collective_id=0),
    )(x2, Wq, Wo, K_ext, V_ext)
    return out.reshape(1, SQ, D_MODEL)


# baseline (device time: 97213 ns/iter reference)
import jax
import jax.numpy as jnp
from jax import lax
from jax.experimental import pallas as pl
from jax.experimental.pallas import tpu as pltpu

N_DEV = 16
H_PER = 8
SQ = 256
SKV = 4096
DH = 128
D_MODEL = 1024
RPC = SQ // N_DEV
SCALE = 0.08838834764831843


def kernel(x, Wq, Wo, K_ext, V_ext):
    x2 = x.reshape(SQ, D_MODEL)

    def body(x_ref, wq_ref, wo_ref, k_hbm, v_hbm, out_ref,
             k_buf, v_buf, attn_ref, partial_ref, recv_ref,
             kv_sems, a_send, a_recv, c_send, c_recv):
        my = lax.axis_index("i")

        barrier = pltpu.get_barrier_semaphore()
        for d in range(1, N_DEV):
            peer = (my + d) % N_DEV
            pl.semaphore_signal(barrier, inc=1, device_id=(peer,),
                                device_id_type=pl.DeviceIdType.MESH)
        pl.semaphore_wait(barrier, N_DEV - 1)

        h0 = my * H_PER
        k_copy = pltpu.make_async_copy(
            k_hbm.at[0, :, pl.ds(h0, H_PER), :], k_buf, kv_sems.at[0])
        v_copy = pltpu.make_async_copy(
            v_hbm.at[0, :, pl.ds(h0, H_PER), :], v_buf, kv_sems.at[1])
        k_copy.start()
        v_copy.start()

        xb = x_ref[...].astype(jnp.bfloat16)
        wqb = wq_ref[...].astype(jnp.bfloat16)
        q = lax.dot_general(xb, wqb, (((1,), (0,)), ((), ())),
                            preferred_element_type=jnp.float32)
        qb = (q * SCALE).astype(jnp.bfloat16)

        k_copy.wait()
        v_copy.wait()

        for h in range(H_PER):
            qh = qb[:, h * DH:(h + 1) * DH]
            kh = k_buf[:, h, :].astype(jnp.bfloat16)
            s = lax.dot_general(qh, kh, (((1,), (1,)), ((), ())),
                                preferred_element_type=jnp.float32)
            m = jnp.max(s, axis=1, keepdims=True)
            p = jnp.exp(s - m)
            l = jnp.sum(p, axis=1, keepdims=True)
            vh = v_buf[:, h, :].astype(jnp.bfloat16)
            o = lax.dot_general(p.astype(jnp.bfloat16), vh,
                                (((1,), (0,)), ((), ())),
                                preferred_element_type=jnp.float32)
            attn_ref[:, h * DH:(h + 1) * DH] = (o / l).astype(jnp.bfloat16)

        wob = wo_ref[...].astype(jnp.bfloat16)
        partial_ref[...] = lax.dot_general(
            attn_ref[...], wob, (((1,), (0,)), ((), ())),
            preferred_element_type=jnp.float32)

        a_sends = []
        for d in range(1, N_DEV):
            peer = (my + d) % N_DEV
            rdma = pltpu.make_async_remote_copy(
                src_ref=partial_ref.at[pl.ds(peer * RPC, RPC), :],
                dst_ref=recv_ref.at[my],
                send_sem=a_send.at[peer],
                recv_sem=a_recv.at[my],
                device_id=(peer,),
                device_id_type=pl.DeviceIdType.MESH,
            )
            rdma.start()
            a_sends.append(rdma)
        for d in range(1, N_DEV):
            src = (my + d) % N_DEV
            pltpu.make_async_remote_copy(
                src_ref=partial_ref.at[pl.ds(0, RPC), :],
                dst_ref=recv_ref.at[src],
                send_sem=a_send.at[src],
                recv_sem=a_recv.at[src],
                device_id=(src,),
                device_id_type=pl.DeviceIdType.MESH,
            ).wait_recv()
        for r in a_sends:
            r.wait_send()

        acc = partial_ref[pl.ds(my * RPC, RPC), :]
        for d in range(1, N_DEV):
            src = (my + d) % N_DEV
            acc = acc + recv_ref[src]
        out_ref[pl.ds(my * RPC, RPC), :] = acc

        c_sends = []
        for d in range(1, N_DEV):
            peer = (my + d) % N_DEV
            rdma = pltpu.make_async_remote_copy(
                src_ref=out_ref.at[pl.ds(my * RPC, RPC), :],
                dst_ref=out_ref.at[pl.ds(my * RPC, RPC), :],
                send_sem=c_send.at[peer],
                recv_sem=c_recv.at[my],
                device_id=(peer,),
                device_id_type=pl.DeviceIdType.MESH,
            )
            rdma.start()
            c_sends.append(rdma)
        for d in range(1, N_DEV):
            src = (my + d) % N_DEV
            pltpu.make_async_remote_copy(
                src_ref=out_ref.at[pl.ds(0, RPC), :],
                dst_ref=out_ref.at[pl.ds(src * RPC, RPC), :],
                send_sem=c_send.at[src],
                recv_sem=c_recv.at[src],
                device_id=(src,),
                device_id_type=pl.DeviceIdType.MESH,
            ).wait_recv()
        for r in c_sends:
            r.wait_send()

    out = pl.pallas_call(
        body,
        out_shape=jax.ShapeDtypeStruct((SQ, D_MODEL), jnp.float32),
        in_specs=[
            pl.BlockSpec(memory_space=pltpu.VMEM),
            pl.BlockSpec(memory_space=pltpu.VMEM),
            pl.BlockSpec(memory_space=pltpu.VMEM),
            pl.BlockSpec(memory_space=pltpu.MemorySpace.HBM),
            pl.BlockSpec(memory_space=pltpu.MemorySpace.HBM),
        ],
        out_specs=pl.BlockSpec(memory_space=pltpu.VMEM),
        scratch_shapes=[
            pltpu.VMEM((SKV, H_PER, DH), jnp.float32),
            pltpu.VMEM((SKV, H_PER, DH), jnp.float32),
            pltpu.VMEM((SQ, D_MODEL), jnp.bfloat16),
            pltpu.VMEM((SQ, D_MODEL), jnp.float32),
            pltpu.VMEM((N_DEV, RPC, D_MODEL), jnp.float32),
            pltpu.SemaphoreType.DMA((2,)),
            pltpu.SemaphoreType.DMA((N_DEV,)),
            pltpu.SemaphoreType.DMA((N_DEV,)),
            pltpu.SemaphoreType.DMA((N_DEV,)),
            pltpu.SemaphoreType.DMA((N_DEV,)),
        ],
        compiler_params=pltpu.CompilerParams(
            collective_id=0, vmem_limit_bytes=64 * 1024 * 1024),
    )(x2, Wq, Wo, K_ext, V_ext)
    return out.reshape(1, SQ, D_MODEL)


# device time: 76354 ns/iter; 1.2732x vs baseline; 1.2732x over previous
import os

import jax
import jax.numpy as jnp
from jax import lax
from jax.experimental import pallas as pl
from jax.experimental.pallas import tpu as pltpu

_NO_COMM = bool(os.environ.get("KBENCH_NO_COMM"))
_NO_COMPUTE = bool(os.environ.get("KBENCH_NO_COMPUTE"))

N_DEV = 16
H_PER = 8
SQ = 256
SKV = 4096
DH = 128
D_MODEL = 1024
RPC = SQ // N_DEV
SCALE = 0.08838834764831843


def _comm(my, partial_ref, recv_ref, out_ref, a_send, a_recv, c_send, c_recv):
    a_sends = []
    for d in range(1, N_DEV):
        peer = (my + d) % N_DEV
        rdma = pltpu.make_async_remote_copy(
            src_ref=partial_ref.at[pl.ds(peer * RPC, RPC), :],
            dst_ref=recv_ref.at[my],
            send_sem=a_send.at[peer],
            recv_sem=a_recv.at[my],
            device_id=(peer,),
            device_id_type=pl.DeviceIdType.MESH,
        )
        rdma.start()
        a_sends.append(rdma)
    for d in range(1, N_DEV):
        src = (my + d) % N_DEV
        pltpu.make_async_remote_copy(
            src_ref=partial_ref.at[pl.ds(0, RPC), :],
            dst_ref=recv_ref.at[src],
            send_sem=a_send.at[src],
            recv_sem=a_recv.at[src],
            device_id=(src,),
            device_id_type=pl.DeviceIdType.MESH,
        ).wait_recv()
    for r in a_sends:
        r.wait_send()

    acc = partial_ref[pl.ds(my * RPC, RPC), :]
    for d in range(1, N_DEV):
        src = (my + d) % N_DEV
        acc = acc + recv_ref[src]
    out_ref[pl.ds(my * RPC, RPC), :] = acc

    c_sends = []
    for d in range(1, N_DEV):
        peer = (my + d) % N_DEV
        rdma = pltpu.make_async_remote_copy(
            src_ref=out_ref.at[pl.ds(my * RPC, RPC), :],
            dst_ref=out_ref.at[pl.ds(my * RPC, RPC), :],
            send_sem=c_send.at[peer],
            recv_sem=c_recv.at[my],
            device_id=(peer,),
            device_id_type=pl.DeviceIdType.MESH,
        )
        rdma.start()
        c_sends.append(rdma)
    for d in range(1, N_DEV):
        src = (my + d) % N_DEV
        pltpu.make_async_remote_copy(
            src_ref=out_ref.at[pl.ds(0, RPC), :],
            dst_ref=out_ref.at[pl.ds(src * RPC, RPC), :],
            send_sem=c_send.at[src],
            recv_sem=c_recv.at[src],
            device_id=(src,),
            device_id_type=pl.DeviceIdType.MESH,
        ).wait_recv()
    for r in c_sends:
        r.wait_send()


def kernel(x, Wq, Wo, K_ext, V_ext):
    x2 = x.reshape(SQ, D_MODEL)

    def body(x_ref, wq_ref, wo_ref, k_hbm, v_hbm, out_ref,
             k_buf, v_buf, attn_ref, partial_ref, recv_ref,
             kv_sems, a_send, a_recv, c_send, c_recv):
        my = lax.axis_index("i")

        barrier = pltpu.get_barrier_semaphore()
        for d in range(1, N_DEV):
            peer = (my + d) % N_DEV
            pl.semaphore_signal(barrier, inc=1, device_id=(peer,),
                                device_id_type=pl.DeviceIdType.MESH)
        pl.semaphore_wait(barrier, N_DEV - 1)

        if _NO_COMPUTE:
            partial_ref[...] = jnp.zeros((SQ, D_MODEL), jnp.float32)
            _comm(my, partial_ref, recv_ref, out_ref,
                  a_send, a_recv, c_send, c_recv)
            return

        h0 = my * H_PER
        k_copy = pltpu.make_async_copy(
            k_hbm.at[0, :, pl.ds(h0, H_PER), :], k_buf, kv_sems.at[0])
        v_copy = pltpu.make_async_copy(
            v_hbm.at[0, :, pl.ds(h0, H_PER), :], v_buf, kv_sems.at[1])
        k_copy.start()
        v_copy.start()

        xb = x_ref[...].astype(jnp.bfloat16)
        wqb = wq_ref[...].astype(jnp.bfloat16)
        q = lax.dot_general(xb, wqb, (((1,), (0,)), ((), ())),
                            preferred_element_type=jnp.float32)
        qb = (q * SCALE).astype(jnp.bfloat16)

        k_copy.wait()
        v_copy.wait()

        for h in range(H_PER):
            qh = qb[:, h * DH:(h + 1) * DH]
            kh = k_buf[:, h, :].astype(jnp.bfloat16)
            s = lax.dot_general(qh, kh, (((1,), (1,)), ((), ())),
                                preferred_element_type=jnp.float32)
            m = jnp.max(s, axis=1, keepdims=True)
            p = jnp.exp(s - m)
            l = jnp.sum(p, axis=1, keepdims=True)
            vh = v_buf[:, h, :].astype(jnp.bfloat16)
            o = lax.dot_general(p.astype(jnp.bfloat16), vh,
                                (((1,), (0,)), ((), ())),
                                preferred_element_type=jnp.float32)
            attn_ref[:, h * DH:(h + 1) * DH] = (o / l).astype(jnp.bfloat16)

        wob = wo_ref[...].astype(jnp.bfloat16)
        partial_ref[...] = lax.dot_general(
            attn_ref[...], wob, (((1,), (0,)), ((), ())),
            preferred_element_type=jnp.float32)

        if _NO_COMM:
            out_ref[...] = partial_ref[...]
        else:
            _comm(my, partial_ref, recv_ref, out_ref,
                  a_send, a_recv, c_send, c_recv)

    out = pl.pallas_call(
        body,
        out_shape=jax.ShapeDtypeStruct((SQ, D_MODEL), jnp.float32),
        in_specs=[
            pl.BlockSpec(memory_space=pltpu.VMEM),
            pl.BlockSpec(memory_space=pltpu.VMEM),
            pl.BlockSpec(memory_space=pltpu.VMEM),
            pl.BlockSpec(memory_space=pltpu.MemorySpace.HBM),
            pl.BlockSpec(memory_space=pltpu.MemorySpace.HBM),
        ],
        out_specs=pl.BlockSpec(memory_space=pltpu.VMEM),
        scratch_shapes=[
            pltpu.VMEM((SKV, H_PER, DH), jnp.float32),
            pltpu.VMEM((SKV, H_PER, DH), jnp.float32),
            pltpu.VMEM((SQ, D_MODEL), jnp.bfloat16),
            pltpu.VMEM((SQ, D_MODEL), jnp.float32),
            pltpu.VMEM((N_DEV, RPC, D_MODEL), jnp.float32),
            pltpu.SemaphoreType.DMA((2,)),
            pltpu.SemaphoreType.DMA((N_DEV,)),
            pltpu.SemaphoreType.DMA((N_DEV,)),
            pltpu.SemaphoreType.DMA((N_DEV,)),
            pltpu.SemaphoreType.DMA((N_DEV,)),
        ],
        compiler_params=pltpu.CompilerParams(
            collective_id=0, vmem_limit_bytes=64 * 1024 * 1024),
    )(x2, Wq, Wo, K_ext, V_ext)
    return out.reshape(1, SQ, D_MODEL)


# device time: 42234 ns/iter; 2.3018x vs baseline; 1.8079x over previous
import os

import jax
import jax.numpy as jnp
from jax import lax
from jax.experimental import pallas as pl
from jax.experimental.pallas import tpu as pltpu

_NO_COMM = bool(os.environ.get("KBENCH_NO_COMM"))
_NO_COMPUTE = bool(os.environ.get("KBENCH_NO_COMPUTE"))
_NO_KV = bool(os.environ.get("KBENCH_NO_KV"))
_NO_SOFTMAX = bool(os.environ.get("KBENCH_NO_SOFTMAX"))
_ATTN_ONLY = bool(os.environ.get("KBENCH_ATTN_ONLY"))
_NO_PV = bool(os.environ.get("KBENCH_NO_PV"))

N_DEV = 16
H_PER = 8
SQ = 256
SKV = 4096
DH = 128
D_MODEL = 1024
RPC = SQ // N_DEV
SCALE = 0.08838834764831843


def _comm(my, partial_ref, recv_ref, out_ref, a_send, a_recv, c_send, c_recv):
    a_sends = []
    for d in range(1, N_DEV):
        peer = (my + d) % N_DEV
        rdma = pltpu.make_async_remote_copy(
            src_ref=partial_ref.at[pl.ds(peer * RPC, RPC), :],
            dst_ref=recv_ref.at[my],
            send_sem=a_send.at[peer],
            recv_sem=a_recv.at[my],
            device_id=(peer,),
            device_id_type=pl.DeviceIdType.MESH,
        )
        rdma.start()
        a_sends.append(rdma)
    for d in range(1, N_DEV):
        src = (my + d) % N_DEV
        pltpu.make_async_remote_copy(
            src_ref=partial_ref.at[pl.ds(0, RPC), :],
            dst_ref=recv_ref.at[src],
            send_sem=a_send.at[src],
            recv_sem=a_recv.at[src],
            device_id=(src,),
            device_id_type=pl.DeviceIdType.MESH,
        ).wait_recv()
    for r in a_sends:
        r.wait_send()

    acc = partial_ref[pl.ds(my * RPC, RPC), :]
    for d in range(1, N_DEV):
        src = (my + d) % N_DEV
        acc = acc + recv_ref[src]
    out_ref[pl.ds(my * RPC, RPC), :] = acc

    c_sends = []
    for d in range(1, N_DEV):
        peer = (my + d) % N_DEV
        rdma = pltpu.make_async_remote_copy(
            src_ref=out_ref.at[pl.ds(my * RPC, RPC), :],
            dst_ref=out_ref.at[pl.ds(my * RPC, RPC), :],
            send_sem=c_send.at[peer],
            recv_sem=c_recv.at[my],
            device_id=(peer,),
            device_id_type=pl.DeviceIdType.MESH,
        )
        rdma.start()
        c_sends.append(rdma)
    for d in range(1, N_DEV):
        src = (my + d) % N_DEV
        pltpu.make_async_remote_copy(
            src_ref=out_ref.at[pl.ds(0, RPC), :],
            dst_ref=out_ref.at[pl.ds(src * RPC, RPC), :],
            send_sem=c_send.at[src],
            recv_sem=c_recv.at[src],
            device_id=(src,),
            device_id_type=pl.DeviceIdType.MESH,
        ).wait_recv()
    for r in c_sends:
        r.wait_send()


def kernel(x, Wq, Wo, K_ext, V_ext):
    x2 = x.reshape(SQ, D_MODEL)

    def body(x_ref, wq_ref, wo_ref, k_hbm, v_hbm, out_ref,
             k_buf, v_buf, attn_ref, partial_ref, recv_ref,
             kv_sems, a_send, a_recv, c_send, c_recv):
        my = lax.axis_index("i")

        barrier = pltpu.get_barrier_semaphore()
        for d in range(1, N_DEV):
            peer = (my + d) % N_DEV
            pl.semaphore_signal(barrier, inc=1, device_id=(peer,),
                                device_id_type=pl.DeviceIdType.MESH)
        pl.semaphore_wait(barrier, N_DEV - 1)

        if _NO_COMPUTE:
            partial_ref[...] = jnp.zeros((SQ, D_MODEL), jnp.float32)
            _comm(my, partial_ref, recv_ref, out_ref,
                  a_send, a_recv, c_send, c_recv)
            return

        h0 = my * H_PER
        k_copy = pltpu.make_async_copy(
            k_hbm.at[0, :, pl.ds(h0, H_PER), :], k_buf, kv_sems.at[0])
        v_copy = pltpu.make_async_copy(
            v_hbm.at[0, :, pl.ds(h0, H_PER), :], v_buf, kv_sems.at[1])
        if not _NO_KV:
            k_copy.start()
            v_copy.start()

        if _ATTN_ONLY:
            qb = x_ref[...].astype(jnp.bfloat16)
        else:
            xb = x_ref[...].astype(jnp.bfloat16)
            wqb = wq_ref[...].astype(jnp.bfloat16)
            q = lax.dot_general(xb, wqb, (((1,), (0,)), ((), ())),
                                preferred_element_type=jnp.float32)
            qb = (q * SCALE).astype(jnp.bfloat16)

        if not _NO_KV:
            k_copy.wait()
            v_copy.wait()

        for h in range(H_PER):
            qh = qb[:, h * DH:(h + 1) * DH]
            kh = k_buf[:, h, :].astype(jnp.bfloat16)
            s = lax.dot_general(qh, kh, (((1,), (1,)), ((), ())),
                                preferred_element_type=jnp.float32)
            if _NO_SOFTMAX:
                p = s
                l = jnp.float32(1.0)
            else:
                m = jnp.max(s, axis=1, keepdims=True)
                p = jnp.exp(s - m)
                l = jnp.sum(p, axis=1, keepdims=True)
            if _NO_PV:
                o = p[:, :DH]
            else:
                vh = v_buf[:, h, :].astype(jnp.bfloat16)
                o = lax.dot_general(p.astype(jnp.bfloat16), vh,
                                    (((1,), (0,)), ((), ())),
                                    preferred_element_type=jnp.float32)
            attn_ref[:, h * DH:(h + 1) * DH] = (o / l).astype(jnp.bfloat16)

        if _ATTN_ONLY:
            partial_ref[...] = attn_ref[...].astype(jnp.float32)
        else:
            wob = wo_ref[...].astype(jnp.bfloat16)
            partial_ref[...] = lax.dot_general(
                attn_ref[...], wob, (((1,), (0,)), ((), ())),
                preferred_element_type=jnp.float32)

        if _NO_COMM:
            out_ref[...] = partial_ref[...]
        else:
            _comm(my, partial_ref, recv_ref, out_ref,
                  a_send, a_recv, c_send, c_recv)

    out = pl.pallas_call(
        body,
        out_shape=jax.ShapeDtypeStruct((SQ, D_MODEL), jnp.float32),
        in_specs=[
            pl.BlockSpec(memory_space=pltpu.VMEM),
            pl.BlockSpec(memory_space=pltpu.VMEM),
            pl.BlockSpec(memory_space=pltpu.VMEM),
            pl.BlockSpec(memory_space=pltpu.MemorySpace.HBM),
            pl.BlockSpec(memory_space=pltpu.MemorySpace.HBM),
        ],
        out_specs=pl.BlockSpec(memory_space=pltpu.VMEM),
        scratch_shapes=[
            pltpu.VMEM((SKV, H_PER, DH), jnp.float32),
            pltpu.VMEM((SKV, H_PER, DH), jnp.float32),
            pltpu.VMEM((SQ, D_MODEL), jnp.bfloat16),
            pltpu.VMEM((SQ, D_MODEL), jnp.float32),
            pltpu.VMEM((N_DEV, RPC, D_MODEL), jnp.float32),
            pltpu.SemaphoreType.DMA((2,)),
            pltpu.SemaphoreType.DMA((N_DEV,)),
            pltpu.SemaphoreType.DMA((N_DEV,)),
            pltpu.SemaphoreType.DMA((N_DEV,)),
            pltpu.SemaphoreType.DMA((N_DEV,)),
        ],
        compiler_params=pltpu.CompilerParams(
            collective_id=0, vmem_limit_bytes=64 * 1024 * 1024),
    )(x2, Wq, Wo, K_ext, V_ext)
    return out.reshape(1, SQ, D_MODEL)


# device time: 41926 ns/iter; 2.3187x vs baseline; 1.0073x over previous
import os

import jax
import jax.numpy as jnp
from jax import lax
from jax.experimental import pallas as pl
from jax.experimental.pallas import tpu as pltpu

_NO_COMM = bool(os.environ.get("KBENCH_NO_COMM"))
_NO_COMPUTE = bool(os.environ.get("KBENCH_NO_COMPUTE"))
_NO_KV = bool(os.environ.get("KBENCH_NO_KV"))
_NO_BARRIER = bool(os.environ.get("KBENCH_NO_BARRIER"))

N_DEV = 16
H_PER = 8
SQ = 256
SKV = 4096
DH = 128
D_MODEL = 1024
RPC = SQ // N_DEV
SCALE = 0.08838834764831843


def _comm(my, partial_ref, pbf_ref, recv_ref, gath_ref, out_ref,
          a_send, a_recv, c_send, c_recv):
    a_sends = []
    for d in range(1, N_DEV):
        peer = (my + d) % N_DEV
        rdma = pltpu.make_async_remote_copy(
            src_ref=pbf_ref.at[pl.ds(peer * RPC, RPC), :],
            dst_ref=recv_ref.at[my],
            send_sem=a_send.at[peer],
            recv_sem=a_recv.at[my],
            device_id=(peer,),
            device_id_type=pl.DeviceIdType.MESH,
        )
        rdma.start()
        a_sends.append(rdma)
    for d in range(1, N_DEV):
        src = (my + d) % N_DEV
        pltpu.make_async_remote_copy(
            src_ref=pbf_ref.at[pl.ds(0, RPC), :],
            dst_ref=recv_ref.at[src],
            send_sem=a_send.at[src],
            recv_sem=a_recv.at[src],
            device_id=(src,),
            device_id_type=pl.DeviceIdType.MESH,
        ).wait_recv()

    acc = partial_ref[pl.ds(my * RPC, RPC), :]
    for d in range(1, N_DEV):
        src = (my + d) % N_DEV
        acc = acc + recv_ref[pl.ds(src, 1)][0].astype(jnp.float32)
    gath_ref[pl.ds(my, 1)] = acc.astype(jnp.bfloat16)[None]

    c_sends = []
    for d in range(1, N_DEV):
        peer = (my + d) % N_DEV
        rdma = pltpu.make_async_remote_copy(
            src_ref=gath_ref.at[my],
            dst_ref=gath_ref.at[my],
            send_sem=c_send.at[peer],
            recv_sem=c_recv.at[my],
            device_id=(peer,),
            device_id_type=pl.DeviceIdType.MESH,
        )
        rdma.start()
        c_sends.append(rdma)
    for d in range(1, N_DEV):
        src = (my + d) % N_DEV
        pltpu.make_async_remote_copy(
            src_ref=gath_ref.at[0],
            dst_ref=gath_ref.at[src],
            send_sem=c_send.at[src],
            recv_sem=c_recv.at[src],
            device_id=(src,),
            device_id_type=pl.DeviceIdType.MESH,
        ).wait_recv()

    out_ref[...] = gath_ref[...].astype(jnp.float32).reshape(SQ, D_MODEL)
    out_ref[pl.ds(my * RPC, RPC), :] = acc

    for r in a_sends:
        r.wait_send()
    for r in c_sends:
        r.wait_send()


def kernel(x, Wq, Wo, K_ext, V_ext):
    x2 = x.reshape(SQ, D_MODEL)

    def body(x_ref, wq_ref, wo_ref, k_hbm, v_hbm, out_ref,
             k_buf, v_buf, attn_ref, partial_ref, pbf_ref,
             recv_ref, gath_ref,
             k_sems, v_sems, a_send, a_recv, c_send, c_recv):
        my = lax.axis_index("i")

        if not _NO_BARRIER:
            barrier = pltpu.get_barrier_semaphore()
            for d in range(1, N_DEV):
                peer = (my + d) % N_DEV
                pl.semaphore_signal(barrier, inc=1, device_id=(peer,),
                                    device_id_type=pl.DeviceIdType.MESH)
            pl.semaphore_wait(barrier, N_DEV - 1)

        if _NO_COMPUTE:
            partial_ref[...] = jnp.zeros((SQ, D_MODEL), jnp.float32)
            pbf_ref[...] = jnp.zeros((SQ, D_MODEL), jnp.bfloat16)
            _comm(my, partial_ref, pbf_ref, recv_ref, gath_ref, out_ref,
                  a_send, a_recv, c_send, c_recv)
            return

        h0 = my * H_PER
        k_copies, v_copies = [], []
        for h in range(H_PER):
            kc = pltpu.make_async_copy(
                k_hbm.at[0, :, h0 + h, :],
                k_buf.at[h], k_sems.at[h])
            vc = pltpu.make_async_copy(
                v_hbm.at[0, :, h0 + h, :],
                v_buf.at[h], v_sems.at[h])
            if not _NO_KV:
                kc.start()
                vc.start()
            k_copies.append(kc)
            v_copies.append(vc)

        xb = x_ref[...].astype(jnp.bfloat16)
        wqb = wq_ref[...].astype(jnp.bfloat16)
        q = lax.dot_general(xb, wqb, (((1,), (0,)), ((), ())),
                            preferred_element_type=jnp.float32)
        qb = (q * SCALE).astype(jnp.bfloat16)

        for h in range(H_PER):
            if not _NO_KV:
                k_copies[h].wait()
                v_copies[h].wait()
            qh = qb[:, h * DH:(h + 1) * DH]
            kh = k_buf[h].astype(jnp.bfloat16)
            s = lax.dot_general(qh, kh, (((1,), (1,)), ((), ())),
                                preferred_element_type=jnp.float32)
            p = jnp.exp(s)
            l = jnp.sum(p, axis=1, keepdims=True)
            vh = v_buf[h].astype(jnp.bfloat16)
            o = lax.dot_general(p.astype(jnp.bfloat16), vh,
                                (((1,), (0,)), ((), ())),
                                preferred_element_type=jnp.float32)
            attn_ref[:, h * DH:(h + 1) * DH] = (o / l).astype(jnp.bfloat16)

        wob = wo_ref[...].astype(jnp.bfloat16)
        partial = lax.dot_general(
            attn_ref[...], wob, (((1,), (0,)), ((), ())),
            preferred_element_type=jnp.float32)
        partial_ref[...] = partial
        pbf_ref[...] = partial.astype(jnp.bfloat16)

        if _NO_COMM:
            out_ref[...] = partial_ref[...]
        else:
            _comm(my, partial_ref, pbf_ref, recv_ref, gath_ref, out_ref,
                  a_send, a_recv, c_send, c_recv)

    out = pl.pallas_call(
        body,
        out_shape=jax.ShapeDtypeStruct((SQ, D_MODEL), jnp.float32),
        in_specs=[
            pl.BlockSpec(memory_space=pltpu.VMEM),
            pl.BlockSpec(memory_space=pltpu.VMEM),
            pl.BlockSpec(memory_space=pltpu.VMEM),
            pl.BlockSpec(memory_space=pltpu.MemorySpace.HBM),
            pl.BlockSpec(memory_space=pltpu.MemorySpace.HBM),
        ],
        out_specs=pl.BlockSpec(memory_space=pltpu.VMEM),
        scratch_shapes=[
            pltpu.VMEM((H_PER, SKV, DH), jnp.float32),
            pltpu.VMEM((H_PER, SKV, DH), jnp.float32),
            pltpu.VMEM((SQ, D_MODEL), jnp.bfloat16),
            pltpu.VMEM((SQ, D_MODEL), jnp.float32),
            pltpu.VMEM((SQ, D_MODEL), jnp.bfloat16),
            pltpu.VMEM((N_DEV, RPC, D_MODEL), jnp.bfloat16),
            pltpu.VMEM((N_DEV, RPC, D_MODEL), jnp.bfloat16),
            pltpu.SemaphoreType.DMA((H_PER,)),
            pltpu.SemaphoreType.DMA((H_PER,)),
            pltpu.SemaphoreType.DMA((N_DEV,)),
            pltpu.SemaphoreType.DMA((N_DEV,)),
            pltpu.SemaphoreType.DMA((N_DEV,)),
            pltpu.SemaphoreType.DMA((N_DEV,)),
        ],
        compiler_params=pltpu.CompilerParams(
            collective_id=None if _NO_BARRIER else 0,
            vmem_limit_bytes=64 * 1024 * 1024),
    )(x2, Wq, Wo, K_ext, V_ext)
    return out.reshape(1, SQ, D_MODEL)
